# Initial kernel scaffold; baseline (speedup 1.0000x reference)
import jax
import jax.numpy as jnp
from jax.experimental import pallas as pl


def kernel(input, target):
    raise NotImplementedError("write your pallas kernel here")


# speedup vs baseline: 21.6017x; 21.6017x over previous
import functools

import jax
import jax.numpy as jnp
from jax import lax
from jax.experimental import pallas as pl
from jax.experimental.pallas import tpu as pltpu
from jax.experimental.pallas import tpu_sc as plsc

NC = 2
NS = 16
L = 16
NW = NC * NS
NIMG = 32
P = 512 * 512
K = 16384
RANGE = 16.0
SCALE = K / RANGE
CH = 8192
NCH = P // CH


def _body(lg_hbm, lb_hbm, out_hbm,
          lg0, lg1, lb0, lb1, cnt, s_sum, outv,
          sg0, sb0, sg1, sb1):
    wid = lax.axis_index("s") * NC + lax.axis_index("c")

    zero = jnp.zeros((L,), jnp.float32)

    def zc(i, _):
        cnt[pl.ds(i * L, L)] = zero
        return 0
    lax.fori_loop(0, 2 * K // L, zc, 0)

    def zs(i, _):
        s_sum[pl.ds(i * L, L)] = zero
        return 0
    lax.fori_loop(0, K // L, zs, 0)

    def start(c, lgbuf, lbbuf, semg, semb):
        pltpu.make_async_copy(lg_hbm.at[wid, pl.ds(c * CH, CH)], lgbuf, semg).start()
        pltpu.make_async_copy(lb_hbm.at[wid, pl.ds(c * CH, CH)], lbbuf, semb).start()

    def wait(c, lgbuf, lbbuf, semg, semb):
        pltpu.make_async_copy(lg_hbm.at[wid, pl.ds(c * CH, CH)], lgbuf, semg).wait()
        pltpu.make_async_copy(lb_hbm.at[wid, pl.ds(c * CH, CH)], lbbuf, semb).wait()

    ones = jnp.ones((L,), jnp.float32)

    def compute_chunk(lgbuf, lbbuf, p_acc):
        def ib(i, p_acc):
            lg = lgbuf[pl.ds(i * L, L)]
            lb = lbbuf[pl.ds(i * L, L)]
            lbf = lb.astype(jnp.float32)
            e = 1.0 - lg * (2.0 * lbf - 1.0)
            msk = e > 0.0
            bkt = jnp.maximum(jnp.minimum((e * SCALE).astype(jnp.int32), K - 1), 0)
            plsc.addupdate_scatter(cnt, [bkt + lb * K], ones, mask=msk)
            plsc.addupdate_scatter(s_sum, [bkt], e, mask=msk)
            return p_acc + lbf
        return lax.fori_loop(0, CH // L, ib, p_acc)

    start(0, lg0, lb0, sg0, sb0)

    def cb(j, p_acc):
        c0 = 2 * j
        start(c0 + 1, lg1, lb1, sg1, sb1)
        wait(c0, lg0, lb0, sg0, sb0)
        p_acc = compute_chunk(lg0, lb0, p_acc)

        @pl.when(j < NCH // 2 - 1)
        def _():
            start(c0 + 2, lg0, lb0, sg0, sb0)

        wait(c0 + 1, lg1, lb1, sg1, sb1)
        p_acc = compute_chunk(lg1, lb1, p_acc)
        return p_acc

    p_acc = lax.fori_loop(0, NCH // 2, cb, jnp.zeros((L,), jnp.float32))
    p = jnp.sum(p_acc)

    def bb(i, carry):
        C, D, acc = carry
        off = K - (i + 1) * L
        cp = lax.rev(cnt[pl.ds(K + off, L)], (0,))
        cn = lax.rev(cnt[pl.ds(off, L)], (0,))
        sv = lax.rev(s_sum[pl.ds(off, L)], (0,))
        cum_c = C + plsc.cumsum(cp)
        cum_d = D + plsc.cumsum(cn)
        c_b = cum_c - cp
        d_b = cum_d - cn
        den_b = p + d_b
        j_before = 1.0 - jnp.where(den_b > 0.0,
                                   (p - c_b) / jnp.where(den_b > 0.0, den_b, 1.0),
                                   1.0)
        den_a = p + cum_d
        j_after = 1.0 - jnp.where(den_a > 0.0,
                                  (p - cum_c) / jnp.where(den_a > 0.0, den_a, 1.0),
                                  1.0)
        tot = cp + cn
        contrib = jnp.where(tot > 0.0,
                            sv * (j_after - j_before) / jnp.where(tot > 0.0, tot, 1.0),
                            0.0)
        return (C + jnp.sum(cp), D + jnp.sum(cn), acc + contrib)

    _, _, acc = lax.fori_loop(
        0, K // L, bb,
        (jnp.float32(0.0), jnp.float32(0.0), jnp.zeros((L,), jnp.float32)))
    loss = jnp.sum(acc)

    outv[...] = jnp.full((L,), loss, jnp.float32)
    pltpu.sync_copy(outv, out_hbm.at[wid])


@jax.jit
def _lovasz_sc(logits, labels):
    mesh = plsc.VectorSubcoreMesh(core_axis_name="c", subcore_axis_name="s")
    k = pl.kernel(
        _body,
        out_type=jax.ShapeDtypeStruct((NW, L), jnp.float32),
        mesh=mesh,
        compiler_params=pltpu.CompilerParams(needs_layout_passes=False),
        scratch_types=[
            pltpu.VMEM((CH,), jnp.float32),
            pltpu.VMEM((CH,), jnp.float32),
            pltpu.VMEM((CH,), jnp.int32),
            pltpu.VMEM((CH,), jnp.int32),
            pltpu.VMEM((2 * K,), jnp.float32),
            pltpu.VMEM((K,), jnp.float32),
            pltpu.VMEM((L,), jnp.float32),
            pltpu.SemaphoreType.DMA,
            pltpu.SemaphoreType.DMA,
            pltpu.SemaphoreType.DMA,
            pltpu.SemaphoreType.DMA,
        ],
    )
    return k(logits, labels)


def kernel(input, target):
    logits = input.reshape(NIMG, P)
    labels = target.reshape(NIMG, P)
    per_image = _lovasz_sc(logits, labels)
    return jnp.mean(per_image[:, 0])


# speedup vs baseline: 22.8810x; 1.0592x over previous
import jax
import jax.numpy as jnp
from jax import lax
from jax.experimental import pallas as pl
from jax.experimental.pallas import tpu as pltpu
from jax.experimental.pallas import tpu_sc as plsc

NC = 2
NS = 16
L = 16
NW = NC * NS
NIMG = 32
P = 512 * 512
K = 16384
RANGE = 16.0
SCALE = K / RANGE
CH = 8192
NCH = P // CH


def _body(lg_hbm, lb_hbm, out_hbm,
          lg0, lg1, lb0, lb1, cnt, s_sum, outv,
          sg0, sb0, sg1, sb1):
    wid = lax.axis_index("s") * NC + lax.axis_index("c")

    zero = jnp.zeros((L,), jnp.float32)

    def zc(i, _):
        cnt[pl.ds(i * L, L)] = zero
        return 0
    lax.fori_loop(0, 2 * K // L, zc, 0, unroll=8)

    def zs(i, _):
        s_sum[pl.ds(i * L, L)] = zero
        return 0
    lax.fori_loop(0, K // L, zs, 0, unroll=8)

    def start(c, lgbuf, lbbuf, semg, semb):
        pltpu.make_async_copy(lg_hbm.at[wid, pl.ds(c * CH, CH)], lgbuf, semg).start()
        pltpu.make_async_copy(lb_hbm.at[wid, pl.ds(c * CH, CH)], lbbuf, semb).start()

    def wait(c, lgbuf, lbbuf, semg, semb):
        pltpu.make_async_copy(lg_hbm.at[wid, pl.ds(c * CH, CH)], lgbuf, semg).wait()
        pltpu.make_async_copy(lb_hbm.at[wid, pl.ds(c * CH, CH)], lbbuf, semb).wait()

    ones = jnp.ones((L,), jnp.float32)

    def compute_chunk(lgbuf, lbbuf):
        def ib(i, _):
            lg = lgbuf[pl.ds(i * L, L)]
            lb = lbbuf[pl.ds(i * L, L)]
            e = jnp.where(lb > 0, 1.0 - lg, 1.0 + lg)
            r = jnp.maximum(e, 0.0)
            bkt = jnp.maximum((K - 1) - (r * SCALE).astype(jnp.int32), 0)
            plsc.addupdate_scatter(cnt, [bkt + lb * K], ones)
            plsc.addupdate_scatter(s_sum, [bkt], r)
            return 0
        lax.fori_loop(0, CH // L, ib, 0, unroll=8)

    start(0, lg0, lb0, sg0, sb0)

    def cb(j, _):
        c0 = 2 * j
        start(c0 + 1, lg1, lb1, sg1, sb1)
        wait(c0, lg0, lb0, sg0, sb0)
        compute_chunk(lg0, lb0)

        @pl.when(j < NCH // 2 - 1)
        def _s():
            start(c0 + 2, lg0, lb0, sg0, sb0)

        wait(c0 + 1, lg1, lb1, sg1, sb1)
        compute_chunk(lg1, lb1)
        return 0

    lax.fori_loop(0, NCH // 2, cb, 0)

    def pp(i, acc):
        return acc + cnt[pl.ds(K + i * L, L)]
    p = jnp.sum(lax.fori_loop(0, K // L, pp, jnp.zeros((L,), jnp.float32),
                              unroll=8))

    def bb(i, carry):
        C, D, acc = carry
        off = i * L
        cp = cnt[pl.ds(K + off, L)]
        cn = cnt[pl.ds(off, L)]
        sv = s_sum[pl.ds(off, L)]
        cum_c = C + plsc.cumsum(cp)
        cum_d = D + plsc.cumsum(cn)
        c_b = cum_c - cp
        d_b = cum_d - cn
        den_b = p + d_b
        j_before = 1.0 - jnp.where(den_b > 0.0,
                                   (p - c_b) / jnp.where(den_b > 0.0, den_b, 1.0),
                                   1.0)
        den_a = p + cum_d
        j_after = 1.0 - jnp.where(den_a > 0.0,
                                  (p - cum_c) / jnp.where(den_a > 0.0, den_a, 1.0),
                                  1.0)
        tot = cp + cn
        contrib = jnp.where(tot > 0.0,
                            sv * (j_after - j_before) / jnp.where(tot > 0.0, tot, 1.0),
                            0.0)
        return (C + jnp.sum(cp), D + jnp.sum(cn), acc + contrib)

    _, _, acc = lax.fori_loop(
        0, K // L, bb,
        (jnp.float32(0.0), jnp.float32(0.0), jnp.zeros((L,), jnp.float32)),
        unroll=2)
    loss = jnp.sum(acc)

    outv[...] = jnp.full((L,), loss, jnp.float32)
    pltpu.sync_copy(outv, out_hbm.at[wid])


@jax.jit
def _lovasz_sc(logits, labels):
    mesh = plsc.VectorSubcoreMesh(core_axis_name="c", subcore_axis_name="s")
    k = pl.kernel(
        _body,
        out_type=jax.ShapeDtypeStruct((NW, L), jnp.float32),
        mesh=mesh,
        compiler_params=pltpu.CompilerParams(needs_layout_passes=False),
        scratch_types=[
            pltpu.VMEM((CH,), jnp.float32),
            pltpu.VMEM((CH,), jnp.float32),
            pltpu.VMEM((CH,), jnp.int32),
            pltpu.VMEM((CH,), jnp.int32),
            pltpu.VMEM((2 * K,), jnp.float32),
            pltpu.VMEM((K,), jnp.float32),
            pltpu.VMEM((L,), jnp.float32),
            pltpu.SemaphoreType.DMA,
            pltpu.SemaphoreType.DMA,
            pltpu.SemaphoreType.DMA,
            pltpu.SemaphoreType.DMA,
        ],
    )
    return k(logits, labels)


def kernel(input, target):
    logits = input.reshape(NIMG, P)
    labels = target.reshape(NIMG, P)
    per_image = _lovasz_sc(logits, labels)
    return jnp.mean(per_image[:, 0])


# speedup vs baseline: 27.0449x; 1.1820x over previous
import jax
import jax.numpy as jnp
from jax import lax
from jax.experimental import pallas as pl
from jax.experimental.pallas import tpu as pltpu
from jax.experimental.pallas import tpu_sc as plsc

NC = 2
NS = 16
L = 16
NW = NC * NS
NIMG = 32
P = 512 * 512
K = 16384
RANGE = 16.0
SCALE = K / RANGE
CH = 8192
NCH = P // CH


def _body(lg_hbm, lb_hbm, out_hbm,
          lg0, lg1, lb0, lb1, cnt, s_sum, outv,
          sg0, sb0, sg1, sb1):
    wid = lax.axis_index("s") * NC + lax.axis_index("c")

    zero = jnp.zeros((L,), jnp.float32)

    def zc(i, _):
        cnt[pl.ds(i * L, L)] = zero
        return 0
    lax.fori_loop(0, 2 * K // L, zc, 0, unroll=8)

    def zs(i, _):
        s_sum[pl.ds(i * L, L)] = zero
        return 0
    lax.fori_loop(0, K // L, zs, 0, unroll=8)

    def start(c, lgbuf, lbbuf, semg, semb):
        off = wid * P + c * CH
        pltpu.make_async_copy(lg_hbm.at[pl.ds(off, CH)], lgbuf, semg).start()
        pltpu.make_async_copy(lb_hbm.at[pl.ds(off, CH)], lbbuf, semb).start()

    def wait(c, lgbuf, lbbuf, semg, semb):
        off = wid * P + c * CH
        pltpu.make_async_copy(lg_hbm.at[pl.ds(off, CH)], lgbuf, semg).wait()
        pltpu.make_async_copy(lb_hbm.at[pl.ds(off, CH)], lbbuf, semb).wait()

    ones = jnp.ones((L,), jnp.float32)

    def compute_chunk(lgbuf, lbbuf):
        def ib(i, _):
            lg = lgbuf[pl.ds(i * L, L)]
            lb = lbbuf[pl.ds(i * L, L)]
            e = jnp.where(lb > 0, 1.0 - lg, 1.0 + lg)
            r = jnp.maximum(e, 0.0)
            bkt = jnp.maximum((K - 1) - (r * SCALE).astype(jnp.int32), 0)
            plsc.addupdate(s_sum.at[pl.ds(0, L)],
                           r + bkt.astype(jnp.float32) * 1e-20)
            return 0
        lax.fori_loop(0, CH // L, ib, 0, unroll=8)

    start(0, lg0, lb0, sg0, sb0)

    def cb(j, _):
        c0 = 2 * j
        start(c0 + 1, lg1, lb1, sg1, sb1)
        wait(c0, lg0, lb0, sg0, sb0)
        compute_chunk(lg0, lb0)

        @pl.when(j < NCH // 2 - 1)
        def _s():
            start(c0 + 2, lg0, lb0, sg0, sb0)

        wait(c0 + 1, lg1, lb1, sg1, sb1)
        compute_chunk(lg1, lb1)
        return 0

    lax.fori_loop(0, NCH // 2, cb, 0)

    def pp(i, acc):
        return acc + cnt[pl.ds(K + i * L, L)]
    p = jnp.sum(lax.fori_loop(0, K // L, pp, jnp.zeros((L,), jnp.float32),
                              unroll=8))

    def bb(i, carry):
        C, D, acc = carry
        off = i * L
        cp = cnt[pl.ds(K + off, L)]
        cn = cnt[pl.ds(off, L)]
        sv = s_sum[pl.ds(off, L)]
        cum_c = C + plsc.cumsum(cp)
        cum_d = D + plsc.cumsum(cn)
        c_b = cum_c - cp
        d_b = cum_d - cn
        den_b = p + d_b
        j_before = 1.0 - jnp.where(den_b > 0.0,
                                   (p - c_b) / jnp.where(den_b > 0.0, den_b, 1.0),
                                   1.0)
        den_a = p + cum_d
        j_after = 1.0 - jnp.where(den_a > 0.0,
                                  (p - cum_c) / jnp.where(den_a > 0.0, den_a, 1.0),
                                  1.0)
        tot = cp + cn
        contrib = jnp.where(tot > 0.0,
                            sv * (j_after - j_before) / jnp.where(tot > 0.0, tot, 1.0),
                            0.0)
        return (C + jnp.sum(cp), D + jnp.sum(cn), acc + contrib)

    _, _, acc = lax.fori_loop(
        0, K // L, bb,
        (jnp.float32(0.0), jnp.float32(0.0), jnp.zeros((L,), jnp.float32)),
        unroll=2)
    loss = jnp.sum(acc)

    outv[...] = jnp.full((L,), loss, jnp.float32)
    pltpu.sync_copy(outv, out_hbm.at[wid])


@jax.jit
def _lovasz_sc(logits, labels):
    mesh = plsc.VectorSubcoreMesh(core_axis_name="c", subcore_axis_name="s")
    k = pl.kernel(
        _body,
        out_type=jax.ShapeDtypeStruct((NW, L), jnp.float32),
        mesh=mesh,
        compiler_params=pltpu.CompilerParams(needs_layout_passes=False,
                                             use_tc_tiling_on_sc=True),
        scratch_types=[
            pltpu.VMEM((CH,), jnp.float32),
            pltpu.VMEM((CH,), jnp.float32),
            pltpu.VMEM((CH,), jnp.int32),
            pltpu.VMEM((CH,), jnp.int32),
            pltpu.VMEM((2 * K,), jnp.float32),
            pltpu.VMEM((K,), jnp.float32),
            pltpu.VMEM((L,), jnp.float32),
            pltpu.SemaphoreType.DMA,
            pltpu.SemaphoreType.DMA,
            pltpu.SemaphoreType.DMA,
            pltpu.SemaphoreType.DMA,
        ],
    )
    return k(logits, labels)


def kernel(input, target):
    logits = input.reshape(NIMG * P)
    labels = target.reshape(NIMG * P)
    per_image = _lovasz_sc(logits, labels)
    return jnp.mean(per_image[:, 0])


# speedup vs baseline: 64.5470x; 2.3867x over previous
import jax
import jax.numpy as jnp
from jax import lax
from jax.experimental import pallas as pl
from jax.experimental.pallas import tpu as pltpu
from jax.experimental.pallas import tpu_sc as plsc

NC = 2
NS = 16
L = 16
NW = NC * NS
NIMG = 32
P = 512 * 512
K = 16384
RANGE = 16.0
SCALE = K / RANGE
DELTA = RANGE / K
MAGIC = float(2 ** 23)
AMAG = MAGIC + SCALE
CI = 0x4B000000 + (K - 1)
CH = 8192
NCH = P // CH


def _body(lg_hbm, lb_hbm, out_hbm,
          lg0, lg1, lb0, lb1, cnt, outv,
          sg0, sb0, sg1, sb1):
    wid = lax.axis_index("s") * NC + lax.axis_index("c")

    zero = jnp.zeros((L,), jnp.float32)

    @plsc.parallel_loop(0, 2 * K // L, unroll=8)
    def _zc(i):
        cnt[pl.ds(i * L, L)] = zero

    def start(c, lgbuf, lbbuf, semg, semb):
        off = wid * P + c * CH
        pltpu.make_async_copy(lg_hbm.at[pl.ds(off, CH)], lgbuf, semg).start()
        pltpu.make_async_copy(lb_hbm.at[pl.ds(off, CH)], lbbuf, semb).start()

    def wait(c, lgbuf, lbbuf, semg, semb):
        off = wid * P + c * CH
        pltpu.make_async_copy(lg_hbm.at[pl.ds(off, CH)], lgbuf, semg).wait()
        pltpu.make_async_copy(lb_hbm.at[pl.ds(off, CH)], lbbuf, semb).wait()

    ones = jnp.ones((L,), jnp.float32)

    def compute_chunk(lgbuf, lbbuf):
        @plsc.parallel_loop(0, CH // L, unroll=8)
        def _ib(i):
            lg = lgbuf[pl.ds(i * L, L)]
            lb = lbbuf[pl.ds(i * L, L)]
            t = lg * SCALE
            w = jnp.where(lb > 0, -t, t)
            u = w + AMAG
            rev = CI - plsc.bitcast(u, jnp.int32)
            rev = jnp.maximum(jnp.minimum(rev, K - 1), 0)
            plsc.addupdate_scatter(cnt, [rev + lb * K], ones)

    start(0, lg0, lb0, sg0, sb0)

    def cb(j, _):
        c0 = 2 * j
        start(c0 + 1, lg1, lb1, sg1, sb1)
        wait(c0, lg0, lb0, sg0, sb0)
        compute_chunk(lg0, lb0)

        @pl.when(j < NCH // 2 - 1)
        def _s():
            start(c0 + 2, lg0, lb0, sg0, sb0)

        wait(c0 + 1, lg1, lb1, sg1, sb1)
        compute_chunk(lg1, lb1)
        return 0

    lax.fori_loop(0, NCH // 2, cb, 0)

    def pp(i, acc):
        return acc + cnt[pl.ds(K + i * L, L)]
    p = jnp.sum(lax.fori_loop(0, K // L, pp, jnp.zeros((L,), jnp.float32),
                              unroll=8))

    last_mask = jnp.arange(L, dtype=jnp.int32) < (L - 1)

    def bb(i, carry):
        C, D, acc = carry
        off = i * L
        cp = cnt[pl.ds(K + off, L)]
        cn = cnt[pl.ds(off, L)]
        cum_c = C + plsc.cumsum(cp)
        cum_d = D + plsc.cumsum(cn)
        den = p + cum_d
        a = jnp.where(den > 0.0,
                      1.0 - (p - cum_c) / jnp.where(den > 0.0, den, 1.0),
                      0.0)
        a = jnp.where((i < K // L - 1) | last_mask, a, 0.0)
        return (C + jnp.sum(cp), D + jnp.sum(cn), acc + a)

    _, _, acc = lax.fori_loop(
        0, K // L, bb,
        (jnp.float32(0.0), jnp.float32(0.0), jnp.zeros((L,), jnp.float32)),
        unroll=2)
    loss = DELTA * (0.5 + jnp.sum(acc))

    outv[...] = jnp.full((L,), loss, jnp.float32)
    pltpu.sync_copy(outv, out_hbm.at[wid])


@jax.jit
def _lovasz_sc(logits, labels):
    mesh = plsc.VectorSubcoreMesh(core_axis_name="c", subcore_axis_name="s")
    k = pl.kernel(
        _body,
        out_type=jax.ShapeDtypeStruct((NW, L), jnp.float32),
        mesh=mesh,
        compiler_params=pltpu.CompilerParams(needs_layout_passes=False),
        scratch_types=[
            pltpu.VMEM((CH,), jnp.float32),
            pltpu.VMEM((CH,), jnp.float32),
            pltpu.VMEM((CH,), jnp.int32),
            pltpu.VMEM((CH,), jnp.int32),
            pltpu.VMEM((2 * K,), jnp.float32),
            pltpu.VMEM((L,), jnp.float32),
            pltpu.SemaphoreType.DMA,
            pltpu.SemaphoreType.DMA,
            pltpu.SemaphoreType.DMA,
            pltpu.SemaphoreType.DMA,
        ],
    )
    return k(logits, labels)


def kernel(input, target):
    logits = input.reshape(NIMG * P)
    labels = target.reshape(NIMG * P)
    per_image = _lovasz_sc(logits, labels)
    return jnp.mean(per_image[:, 0])


# speedup vs baseline: 105.4768x; 1.6341x over previous
import jax
import jax.numpy as jnp
from jax import lax
from jax.experimental import pallas as pl
from jax.experimental.pallas import tpu as pltpu
from jax.experimental.pallas import tpu_sc as plsc

NC = 2
NS = 16
L = 16
NW = NC * NS
NIMG = 32
P = 512 * 512
K = 16384
RANGE = 16.0
SCALE = K / RANGE
DELTA = RANGE / K
MAGIC = float(2 ** 23)
AMAG = MAGIC + SCALE
CI = 0x4B000000 + (K - 1)
CH = 8192
NCH = P // CH


def _body(lg_hbm, lb_hbm, out_hbm,
          lg0, lg1, lb0, lb1, cnt, outv,
          sg0, sb0, sg1, sb1):
    wid = lax.axis_index("s") * NC + lax.axis_index("c")

    zero = jnp.zeros((L,), jnp.float32)

    @plsc.parallel_loop(0, 2 * K // L, unroll=8)
    def _zc(i):
        cnt[pl.ds(i * L, L)] = zero

    ROWS = CH // 512

    def start(c, lgbuf, lbbuf, semg, semb):
        r0 = c * ROWS
        pltpu.make_async_copy(lg_hbm.at[wid, pl.ds(r0, ROWS), :], lgbuf, semg).start()
        pltpu.make_async_copy(lb_hbm.at[wid, pl.ds(r0, ROWS), :], lbbuf, semb).start()

    def wait(c, lgbuf, lbbuf, semg, semb):
        r0 = c * ROWS
        pltpu.make_async_copy(lg_hbm.at[wid, pl.ds(r0, ROWS), :], lgbuf, semg).wait()
        pltpu.make_async_copy(lb_hbm.at[wid, pl.ds(r0, ROWS), :], lbbuf, semb).wait()

    ones = jnp.ones((L,), jnp.float32)

    def compute_chunk(lgbuf, lbbuf):
        @plsc.parallel_loop(0, CH // L, unroll=8)
        def _ib(i):
            r = i >> 5
            c = (i & 31) * L
            lg = lgbuf[r, pl.ds(c, L)]
            lb = lbbuf[r, pl.ds(c, L)]
            t = lg * SCALE
            w = jnp.where(lb > 0, -t, t)
            u = w + AMAG
            rev = CI - plsc.bitcast(u, jnp.int32)
            rev = jnp.maximum(jnp.minimum(rev, K - 1), 0)
            plsc.addupdate_scatter(cnt, [rev + lb * K], ones)

    start(0, lg0, lb0, sg0, sb0)

    def cb(j, _):
        c0 = 2 * j
        start(c0 + 1, lg1, lb1, sg1, sb1)
        wait(c0, lg0, lb0, sg0, sb0)
        compute_chunk(lg0, lb0)

        @pl.when(j < NCH // 2 - 1)
        def _s():
            start(c0 + 2, lg0, lb0, sg0, sb0)

        wait(c0 + 1, lg1, lb1, sg1, sb1)
        compute_chunk(lg1, lb1)
        return 0

    lax.fori_loop(0, NCH // 2, cb, 0)

    def pp(i, acc):
        return acc + cnt[pl.ds(K + i * L, L)]
    p = jnp.sum(lax.fori_loop(0, K // L, pp, jnp.zeros((L,), jnp.float32),
                              unroll=8))

    last_mask = jnp.arange(L, dtype=jnp.int32) < (L - 1)

    def bb(i, carry):
        C, D, acc = carry
        off = i * L
        cp = cnt[pl.ds(K + off, L)]
        cn = cnt[pl.ds(off, L)]
        cum_c = C + plsc.cumsum(cp)
        cum_d = D + plsc.cumsum(cn)
        den = p + cum_d
        a = jnp.where(den > 0.0,
                      1.0 - (p - cum_c) / jnp.where(den > 0.0, den, 1.0),
                      0.0)
        a = jnp.where((i < K // L - 1) | last_mask, a, 0.0)
        return (C + jnp.sum(cp), D + jnp.sum(cn), acc + a)

    _, _, acc = lax.fori_loop(
        0, K // L, bb,
        (jnp.float32(0.0), jnp.float32(0.0), jnp.zeros((L,), jnp.float32)),
        unroll=2)
    loss = DELTA * (0.5 + jnp.sum(acc))

    outv[...] = jnp.full((L,), loss, jnp.float32)
    pltpu.sync_copy(outv, out_hbm.at[wid])


@jax.jit
def _lovasz_sc(logits, labels):
    mesh = plsc.VectorSubcoreMesh(core_axis_name="c", subcore_axis_name="s")
    k = pl.kernel(
        _body,
        out_type=jax.ShapeDtypeStruct((NW, L), jnp.float32),
        mesh=mesh,
        compiler_params=pltpu.CompilerParams(needs_layout_passes=False,
                                             use_tc_tiling_on_sc=True),
        scratch_types=[
            pltpu.VMEM((CH // 512, 512), jnp.float32),
            pltpu.VMEM((CH // 512, 512), jnp.float32),
            pltpu.VMEM((CH // 512, 512), jnp.int32),
            pltpu.VMEM((CH // 512, 512), jnp.int32),
            pltpu.VMEM((2 * K,), jnp.float32),
            pltpu.VMEM((L,), jnp.float32),
            pltpu.SemaphoreType.DMA,
            pltpu.SemaphoreType.DMA,
            pltpu.SemaphoreType.DMA,
            pltpu.SemaphoreType.DMA,
        ],
    )
    return k(logits, labels)


def kernel(input, target):
    logits = jnp.squeeze(input, axis=1)
    labels = jnp.squeeze(target, axis=1)
    per_image = _lovasz_sc(logits, labels)
    return jnp.mean(per_image[:, 0])
